# baseline (speedup 1.0000x reference)
import jax
import jax.numpy as jnp
from jax.experimental import pallas as pl
from jax.experimental.pallas import tpu as pltpu

_CHUNK = 64


def _body(m_ref, mn_ref, x_hbm, o_ref, scratch, sems):
    b = pl.program_id(0)
    nb = pl.num_programs(0)
    _, S, D = scratch.shape
    nc = S // _CHUNK

    length = jnp.sum(m_ref[0, 0, :])
    length_nxt = jnp.sum(mn_ref[0, 0, :])

    def chunk_copy(row, buf, c):
        return pltpu.make_async_copy(
            x_hbm.at[row, pl.ds(c * _CHUNK, _CHUNK), :],
            scratch.at[buf, pl.ds(c * _CHUNK, _CHUNK), :],
            sems.at[buf],
        )

    def issue(row, buf, row_len):
        nch = (row_len + _CHUNK - 1) // _CHUNK

        def st(c, carry):
            @pl.when(c < nch)
            def _():
                chunk_copy(row, buf, c).start()
            return carry

        jax.lax.fori_loop(0, nc, st, 0, unroll=True)

    def wait_row(row, buf, row_len):
        nch = (row_len + _CHUNK - 1) // _CHUNK

        def wt(c, carry):
            @pl.when(c < nch)
            def _():
                chunk_copy(row, buf, c).wait()
            return carry

        jax.lax.fori_loop(0, nc, wt, 0, unroll=True)

    @pl.when(b == 0)
    def _():
        issue(b, 0, length)

    nxt = b + 1

    @pl.when((nxt < nb) & (nxt % 2 == 0))
    def _():
        issue(nxt, 0, length_nxt)

    @pl.when((nxt < nb) & (nxt % 2 == 1))
    def _():
        issue(nxt, 1, length_nxt)

    @pl.when(b % 2 == 0)
    def _():
        wait_row(b, 0, length)

    @pl.when(b % 2 == 1)
    def _():
        wait_row(b, 1, length)

    zeros_c = jnp.zeros((_CHUNK, D), dtype=o_ref.dtype)
    for buf in (0, 1):

        @pl.when(b % 2 == buf)
        def _(buf=buf):
            for c in range(nc):
                lo = c * _CHUNK

                @pl.when(lo < length)
                def _(lo=lo):
                    pos = jax.lax.broadcasted_iota(jnp.int32, (_CHUNK, 1), 0) + lo
                    m_t = m_ref[0, 0, pl.ds(lo, _CHUNK)][:, None]
                    keep = (pos < length) & (m_t > 0)
                    o_ref[0, pl.ds(lo, _CHUNK), :] = jnp.where(
                        keep, scratch[buf, pl.ds(lo, _CHUNK), :], zeros_c
                    )

                @pl.when(lo >= length)
                def _(lo=lo):
                    o_ref[0, pl.ds(lo, _CHUNK), :] = zeros_c


def kernel(x, mask):
    B, S, D = x.shape
    m3 = mask.astype(jnp.int32).reshape(B, 1, S)
    return pl.pallas_call(
        _body,
        grid=(B,),
        in_specs=[
            pl.BlockSpec((1, 1, S), lambda b: (b, 0, 0)),
            pl.BlockSpec((1, 1, S), lambda b: (jnp.minimum(b + 1, B - 1), 0, 0)),
            pl.BlockSpec(memory_space=pl.ANY),
        ],
        out_specs=pl.BlockSpec((1, S, D), lambda b: (b, 0, 0)),
        out_shape=jax.ShapeDtypeStruct((B, S, D), x.dtype),
        scratch_shapes=[
            pltpu.VMEM((2, S, D), x.dtype),
            pltpu.SemaphoreType.DMA((2,)),
        ],
    )(m3, m3, x)
